# Initial kernel scaffold; baseline (speedup 1.0000x reference)
import functools

import jax
import jax.numpy as jnp
from jax import lax
from jax.experimental import pallas as pl
from jax.experimental.pallas import tpu as pltpu
from jax.experimental.pallas import tpu_sc as plsc

N = 10000
E = 160000
NSUB = 16
NCORE = 2
EB = 125
EPT = E // NSUB
NBE = EPT // EB
RPT = N // NSUB
R = 2000
GRID = N // R


def _make_agg(nslab, w):
  rounds = nslab // NCORE
  mesh = plsc.VectorSubcoreMesh(core_axis_name="c", subcore_axis_name="s")

  @functools.partial(
      pl.kernel,
      mesh=mesh,
      out_type=jax.ShapeDtypeStruct((nslab, N, w), jnp.float32),
      scratch_types=[
          pltpu.VMEM((NBE, EB), jnp.int32),
          pltpu.VMEM((NBE, EB), jnp.int32),
          pltpu.VMEM((2, EB, w), jnp.float32),
          pltpu.VMEM_SHARED((N, w), jnp.float32),
          pltpu.SemaphoreType.DMA,
          pltpu.SemaphoreType.DMA,
      ],
  )
  def agg(u_hbm, src_hbm, dst_hbm, out_hbm, src_v, dst_v, rows_v, acc_sh,
          sem0, sem1):
    c = lax.axis_index("c")
    s = lax.axis_index("s")
    pltpu.sync_copy(src_hbm.at[s], src_v)
    pltpu.sync_copy(dst_hbm.at[s], dst_v)
    sems = (sem0, sem1)

    for r in range(rounds):
      j = r * NCORE + c
      pltpu.sync_copy(u_hbm.at[j, pl.ds(s * RPT, RPT)],
                      acc_sh.at[pl.ds(s * RPT, RPT)])
      plsc.subcore_barrier()

      def start(b, q):
        pltpu.async_copy(u_hbm.at[j].at[src_v.at[b]], rows_v.at[q], sems[q])

      def wait(b, q):
        pltpu.make_async_copy(u_hbm.at[j].at[src_v.at[b]], rows_v.at[q],
                              sems[q]).wait()

      start(0, 0)

      def body(i, carry):
        for q in range(2):
          b = i * 2 + q

          @pl.when(b + 1 < NBE)
          def _():
            start(b + 1, (q + 1) % 2)

          wait(b, q)
          pltpu.sync_copy(rows_v.at[q], acc_sh.at[dst_v.at[b]], add=True)
        return carry

      lax.fori_loop(0, NBE // 2, body, 0)
      plsc.subcore_barrier()
      pltpu.sync_copy(acc_sh.at[pl.ds(s * RPT, RPT)],
                      out_hbm.at[j, pl.ds(s * RPT, RPT)])
      plsc.subcore_barrier()

  return agg


_agg2_128 = _make_agg(2, 128)
_agg4_128 = _make_agg(4, 128)
_agg2_16 = _make_agg(2, 16)


def _t0_body(hist_ref, x_ref, dinvb_ref, u0_ref):
  deg = hist_ref[0, :, 0:1]
  dinvb = jnp.broadcast_to(lax.rsqrt(deg), (R, 128))
  dinvb_ref[...] = dinvb
  for j in range(2):
    u0_ref[j] = x_ref[:, j * 128:(j + 1) * 128] * dinvb


def _t0(hist, x):
  return pl.pallas_call(
      _t0_body,
      grid=(GRID,),
      in_specs=[
          pl.BlockSpec((1, R, 16), lambda i: (0, i, 0)),
          pl.BlockSpec((R, 256), lambda i: (i, 0)),
      ],
      out_specs=[
          pl.BlockSpec((R, 128), lambda i: (i, 0)),
          pl.BlockSpec((2, R, 128), lambda i: (0, i, 0)),
      ],
      out_shape=[
          jax.ShapeDtypeStruct((N, 128), jnp.float32),
          jax.ShapeDtypeStruct((2, N, 128), jnp.float32),
      ],
  )(hist, x)


def _mm_body(dinvb_ref, in_ref, w_ref, b_ref, out_ref, *, kslab, oslab):
  acc = jnp.zeros((R, oslab * 128), jnp.float32)
  dinvb = dinvb_ref[...]
  for kj in range(kslab):
    blk = in_ref[kj] * dinvb
    acc += jnp.dot(blk, w_ref[pl.ds(kj * 128, 128), :],
                   preferred_element_type=jnp.float32)
  acc = jnp.maximum(acc + b_ref[...], 0.0)
  for oj in range(oslab):
    out_ref[oj] = acc[:, oj * 128:(oj + 1) * 128] * dinvb


def _mm(dinvb, s_in, w, b, kslab, oslab):
  body = functools.partial(_mm_body, kslab=kslab, oslab=oslab)
  return pl.pallas_call(
      body,
      grid=(GRID,),
      in_specs=[
          pl.BlockSpec((R, 128), lambda i: (i, 0)),
          pl.BlockSpec((kslab, R, 128), lambda i: (0, i, 0)),
          pl.BlockSpec((kslab * 128, oslab * 128), lambda i: (0, 0)),
          pl.BlockSpec((1, oslab * 128), lambda i: (0, 0)),
      ],
      out_specs=pl.BlockSpec((oslab, R, 128), lambda i: (0, i, 0)),
      out_shape=jax.ShapeDtypeStruct((oslab, N, 128), jnp.float32),
  )(dinvb, s_in, w, b)


def _t3_body(dinvb_ref, in_ref, w2_ref, b2_ref, wmu_ref, wls_ref, out_ref):
  acc = jnp.zeros((R, 512), jnp.float32)
  dinvb = dinvb_ref[...]
  for kj in range(4):
    blk = in_ref[kj] * dinvb
    acc += jnp.dot(blk, w2_ref[pl.ds(kj * 128, 128), :],
                   preferred_element_type=jnp.float32)
  h3 = jnp.maximum(acc + b2_ref[...], 0.0)
  out_ref[0] = jnp.dot(h3, wmu_ref[...],
                       preferred_element_type=jnp.float32) * dinvb
  out_ref[1] = jnp.dot(h3, wls_ref[...],
                       preferred_element_type=jnp.float32) * dinvb


def _t3(dinvb, s2, w2, b2, wmu, wls):
  return pl.pallas_call(
      _t3_body,
      grid=(GRID,),
      in_specs=[
          pl.BlockSpec((R, 128), lambda i: (i, 0)),
          pl.BlockSpec((4, R, 128), lambda i: (0, i, 0)),
          pl.BlockSpec((512, 512), lambda i: (0, 0)),
          pl.BlockSpec((1, 512), lambda i: (0, 0)),
          pl.BlockSpec((512, 128), lambda i: (0, 0)),
          pl.BlockSpec((512, 128), lambda i: (0, 0)),
      ],
      out_specs=pl.BlockSpec((2, R, 128), lambda i: (0, i, 0)),
      out_shape=jax.ShapeDtypeStruct((2, N, 128), jnp.float32),
  )(dinvb, s2, w2, b2, wmu, wls)


def _t4_body(dinvb_ref, s3_ref, bmu_ref, bls_ref, mu_ref, ls_ref):
  dinvb = dinvb_ref[...]
  mu_ref[...] = s3_ref[0] * dinvb + bmu_ref[...]
  ls_ref[...] = s3_ref[1] * dinvb + bls_ref[...]


def _t4(dinvb, s3, bmu, bls):
  return pl.pallas_call(
      _t4_body,
      grid=(GRID,),
      in_specs=[
          pl.BlockSpec((R, 128), lambda i: (i, 0)),
          pl.BlockSpec((2, R, 128), lambda i: (0, i, 0)),
          pl.BlockSpec((1, 128), lambda i: (0, 0)),
          pl.BlockSpec((1, 128), lambda i: (0, 0)),
      ],
      out_specs=[
          pl.BlockSpec((R, 128), lambda i: (i, 0)),
          pl.BlockSpec((R, 128), lambda i: (i, 0)),
      ],
      out_shape=[
          jax.ShapeDtypeStruct((N, 128), jnp.float32),
          jax.ShapeDtypeStruct((N, 128), jnp.float32),
      ],
  )(dinvb, s3, bmu, bls)


@jax.jit
def kernel(x, edge_index, W0, b0, W1, b1, W2, b2, W_mu, b_mu, W_ls, b_ls):
  src = edge_index[0].reshape(NSUB, NBE, EB)
  dst = edge_index[1].reshape(NSUB, NBE, EB)

  ones = jnp.ones((2, N, 16), jnp.float32)
  hist = _agg2_16(ones, src, dst)
  dinvb, u0 = _t0(hist, x)
  s0 = _agg2_128(u0, src, dst)
  u1 = _mm(dinvb, s0, W0, b0.reshape(1, 512), 2, 4)
  s1 = _agg4_128(u1, src, dst)
  u2 = _mm(dinvb, s1, W1, b1.reshape(1, 512), 4, 4)
  s2 = _agg4_128(u2, src, dst)
  t = _t3(dinvb, s2, W2, b2.reshape(1, 512), W_mu, W_ls)
  s3 = _agg2_128(t, src, dst)
  mu, logstd = _t4(dinvb, s3, b_mu.reshape(1, 128), b_ls.reshape(1, 128))
  return (mu, logstd)


# speedup vs baseline: 14.2411x; 14.2411x over previous
import functools

import jax
import jax.numpy as jnp
from jax import lax
from jax.experimental import pallas as pl
from jax.experimental.pallas import tpu as pltpu
from jax.experimental.pallas import tpu_sc as plsc

N = 10000
E = 160000
NSUB = 16
NCORE = 2
EB = 125
EPT = E // NSUB
NBE = EPT // EB
CHB = 10
NCH = NBE // CHB
RS = 624
REM = N - NSUB * RS
R = 2000
GRID = N // R


def _make_agg(nslab, w):
  rounds = nslab // NCORE
  mesh = plsc.VectorSubcoreMesh(core_axis_name="c", subcore_axis_name="s")

  @functools.partial(
      pl.kernel,
      mesh=mesh,
      out_type=jax.ShapeDtypeStruct((nslab, N, w), jnp.float32),
      scratch_types=[
          pltpu.VMEM((CHB, EB), jnp.int32),
          pltpu.VMEM((CHB, EB), jnp.int32),
          pltpu.VMEM((2, EB, w), jnp.float32),
          pltpu.VMEM_SHARED((N, w), jnp.float32),
          pltpu.SemaphoreType.DMA,
          pltpu.SemaphoreType.DMA,
      ],
  )
  def agg(u_hbm, src_hbm, dst_hbm, out_hbm, src_v, dst_v, rows_v, acc_sh,
          sem0, sem1):
    c = lax.axis_index("c")
    s = lax.axis_index("s")
    sems = (sem0, sem1)

    for r in range(rounds):
      j = r * NCORE + c
      off = pl.multiple_of(s * RS, 8)
      pltpu.sync_copy(u_hbm.at[j, pl.ds(off, RS)],
                      acc_sh.at[pl.ds(off, RS)])

      @pl.when(s == 0)
      def _():
        pltpu.sync_copy(u_hbm.at[j, pl.ds(NSUB * RS, REM)],
                        acc_sh.at[pl.ds(NSUB * RS, REM)])

      plsc.subcore_barrier()

      def start(b, q):
        pltpu.async_copy(u_hbm.at[j].at[src_v.at[b]], rows_v.at[q], sems[q])

      def wait(b, q):
        pltpu.make_async_copy(u_hbm.at[j].at[src_v.at[b]], rows_v.at[q],
                              sems[q]).wait()

      for ch in range(NCH):
        pltpu.sync_copy(src_hbm.at[s, ch], src_v)
        pltpu.sync_copy(dst_hbm.at[s, ch], dst_v)
        start(0, 0)

        def body(i, carry):
          for q in range(2):
            b = i * 2 + q

            @pl.when(b + 1 < CHB)
            def _():
              start(b + 1, (q + 1) % 2)

            wait(b, q)
            pltpu.sync_copy(rows_v.at[q], acc_sh.at[dst_v.at[b]], add=True)
          return carry

        lax.fori_loop(0, CHB // 2, body, 0)
      plsc.subcore_barrier()
      pltpu.sync_copy(acc_sh.at[pl.ds(off, RS)],
                      out_hbm.at[j, pl.ds(off, RS)])

      @pl.when(s == 0)
      def _():
        pltpu.sync_copy(acc_sh.at[pl.ds(NSUB * RS, REM)],
                        out_hbm.at[j, pl.ds(NSUB * RS, REM)])

      plsc.subcore_barrier()

  return agg


_agg2_128 = _make_agg(2, 128)
_agg4_128 = _make_agg(4, 128)


def _t0_body(hist_ref, x_ref, dinvb_ref, u0_ref):
  deg = hist_ref[0, :, 0:1]
  dinvb = jnp.broadcast_to(lax.rsqrt(deg), (R, 128))
  dinvb_ref[...] = dinvb
  for j in range(2):
    u0_ref[j] = x_ref[:, j * 128:(j + 1) * 128] * dinvb


def _t0(hist, x):
  return pl.pallas_call(
      _t0_body,
      grid=(GRID,),
      in_specs=[
          pl.BlockSpec((1, R, 128), lambda i: (0, i, 0)),
          pl.BlockSpec((R, 256), lambda i: (i, 0)),
      ],
      out_specs=[
          pl.BlockSpec((R, 128), lambda i: (i, 0)),
          pl.BlockSpec((2, R, 128), lambda i: (0, i, 0)),
      ],
      out_shape=[
          jax.ShapeDtypeStruct((N, 128), jnp.float32),
          jax.ShapeDtypeStruct((2, N, 128), jnp.float32),
      ],
  )(hist, x)


def _mm_body(dinvb_ref, in_ref, w_ref, b_ref, out_ref, *, kslab, oslab):
  acc = jnp.zeros((R, oslab * 128), jnp.float32)
  dinvb = dinvb_ref[...]
  for kj in range(kslab):
    blk = in_ref[kj] * dinvb
    acc += jnp.dot(blk, w_ref[pl.ds(kj * 128, 128), :],
                   preferred_element_type=jnp.float32)
  acc = jnp.maximum(acc + b_ref[...], 0.0)
  for oj in range(oslab):
    out_ref[oj] = acc[:, oj * 128:(oj + 1) * 128] * dinvb


def _mm(dinvb, s_in, w, b, kslab, oslab):
  body = functools.partial(_mm_body, kslab=kslab, oslab=oslab)
  return pl.pallas_call(
      body,
      grid=(GRID,),
      in_specs=[
          pl.BlockSpec((R, 128), lambda i: (i, 0)),
          pl.BlockSpec((kslab, R, 128), lambda i: (0, i, 0)),
          pl.BlockSpec((kslab * 128, oslab * 128), lambda i: (0, 0)),
          pl.BlockSpec((1, oslab * 128), lambda i: (0, 0)),
      ],
      out_specs=pl.BlockSpec((oslab, R, 128), lambda i: (0, i, 0)),
      out_shape=jax.ShapeDtypeStruct((oslab, N, 128), jnp.float32),
  )(dinvb, s_in, w, b)


def _t3_body(dinvb_ref, in_ref, w2_ref, b2_ref, wmu_ref, wls_ref, out_ref):
  acc = jnp.zeros((R, 512), jnp.float32)
  dinvb = dinvb_ref[...]
  for kj in range(4):
    blk = in_ref[kj] * dinvb
    acc += jnp.dot(blk, w2_ref[pl.ds(kj * 128, 128), :],
                   preferred_element_type=jnp.float32)
  h3 = jnp.maximum(acc + b2_ref[...], 0.0)
  out_ref[0] = jnp.dot(h3, wmu_ref[...],
                       preferred_element_type=jnp.float32) * dinvb
  out_ref[1] = jnp.dot(h3, wls_ref[...],
                       preferred_element_type=jnp.float32) * dinvb


def _t3(dinvb, s2, w2, b2, wmu, wls):
  return pl.pallas_call(
      _t3_body,
      grid=(GRID,),
      in_specs=[
          pl.BlockSpec((R, 128), lambda i: (i, 0)),
          pl.BlockSpec((4, R, 128), lambda i: (0, i, 0)),
          pl.BlockSpec((512, 512), lambda i: (0, 0)),
          pl.BlockSpec((1, 512), lambda i: (0, 0)),
          pl.BlockSpec((512, 128), lambda i: (0, 0)),
          pl.BlockSpec((512, 128), lambda i: (0, 0)),
      ],
      out_specs=pl.BlockSpec((2, R, 128), lambda i: (0, i, 0)),
      out_shape=jax.ShapeDtypeStruct((2, N, 128), jnp.float32),
  )(dinvb, s2, w2, b2, wmu, wls)


def _t4_body(dinvb_ref, s3_ref, bmu_ref, bls_ref, mu_ref, ls_ref):
  dinvb = dinvb_ref[...]
  mu_ref[...] = s3_ref[0] * dinvb + bmu_ref[...]
  ls_ref[...] = s3_ref[1] * dinvb + bls_ref[...]


def _t4(dinvb, s3, bmu, bls):
  return pl.pallas_call(
      _t4_body,
      grid=(GRID,),
      in_specs=[
          pl.BlockSpec((R, 128), lambda i: (i, 0)),
          pl.BlockSpec((2, R, 128), lambda i: (0, i, 0)),
          pl.BlockSpec((1, 128), lambda i: (0, 0)),
          pl.BlockSpec((1, 128), lambda i: (0, 0)),
      ],
      out_specs=[
          pl.BlockSpec((R, 128), lambda i: (i, 0)),
          pl.BlockSpec((R, 128), lambda i: (i, 0)),
      ],
      out_shape=[
          jax.ShapeDtypeStruct((N, 128), jnp.float32),
          jax.ShapeDtypeStruct((N, 128), jnp.float32),
      ],
  )(dinvb, s3, bmu, bls)


@jax.jit
def kernel(x, edge_index, W0, b0, W1, b1, W2, b2, W_mu, b_mu, W_ls, b_ls):
  src = edge_index[0].reshape(NSUB, NCH, CHB, EB)
  dst = edge_index[1].reshape(NSUB, NCH, CHB, EB)

  ones = jnp.ones((2, N, 128), jnp.float32)
  hist = _agg2_128(ones, src, dst)
  dinvb, u0 = _t0(hist, x)
  s0 = _agg2_128(u0, src, dst)
  u1 = _mm(dinvb, s0, W0, b0.reshape(1, 512), 2, 4)
  s1 = _agg4_128(u1, src, dst)
  u2 = _mm(dinvb, s1, W1, b1.reshape(1, 512), 4, 4)
  s2 = _agg4_128(u2, src, dst)
  t = _t3(dinvb, s2, W2, b2.reshape(1, 512), W_mu, W_ls)
  s3 = _agg2_128(t, src, dst)
  mu, logstd = _t4(dinvb, s3, b_mu.reshape(1, 128), b_ls.reshape(1, 128))
  return (mu, logstd)
